# Initial kernel scaffold; baseline (speedup 1.0000x reference)
import jax
import jax.numpy as jnp
from jax.experimental import pallas as pl


def kernel(h_v, edge_index, h_e, params):
    raise NotImplementedError("write your pallas kernel here")


# speedup vs baseline: 6.8137x; 6.8137x over previous
import functools

import numpy as np
import jax
import jax.numpy as jnp
from jax import lax
from jax.experimental import pallas as pl
from jax.experimental.pallas import tpu as pltpu
from jax.experimental.pallas import tpu_sc as plsc

_NC = 2
_NS = 16
_L = 16
_CH = 128


def _sc_edge_pass(a_tab, b_tab, c8, src, dst, with_counts):
    n = a_tab.shape[0]
    e8 = c8.shape[0]
    e = e8 * 8
    nw = _NC * _NS
    nchunk = e // _CH
    chp = _CH // 8
    zr = 640
    npad = zr * _NS
    per_w = (nchunk + nw - 1) // nw

    out_type = [
        jax.ShapeDtypeStruct((e8, 8 * _L), jnp.float32),
        jax.ShapeDtypeStruct((_NC * npad, _L), jnp.float32),
    ]
    scratch = [
        pltpu.VMEM((_CH,), jnp.int32),
        pltpu.VMEM((_CH,), jnp.int32),
        pltpu.VMEM((_CH, _L), jnp.float32),
        pltpu.VMEM((_CH, _L), jnp.float32),
        pltpu.VMEM((chp, 8 * _L), jnp.float32),
        pltpu.VMEM((_CH, _L), jnp.float32),
        pltpu.VMEM((chp, 8 * _L), jnp.float32),
        pltpu.VMEM((zr, _L), jnp.float32),
        pltpu.VMEM_SHARED((npad, _L), jnp.float32),
        pltpu.SemaphoreType.DMA,
        pltpu.SemaphoreType.DMA,
    ]
    if with_counts:
        out_type.append(jax.ShapeDtypeStruct((_NC * npad, _L), jnp.float32))
        scratch.append(pltpu.VMEM((_CH, _L), jnp.float32))
        scratch.append(pltpu.VMEM_SHARED((npad, _L), jnp.float32))

    mesh = plsc.VectorSubcoreMesh(core_axis_name="c", subcore_axis_name="s")

    @functools.partial(
        pl.kernel, out_type=tuple(out_type), mesh=mesh,
        scratch_types=scratch,
        compiler_params=pltpu.CompilerParams(use_tc_tiling_on_sc=False))
    def sc_kernel(a_hbm, b_hbm, c_hbm, src_hbm, dst_hbm, eo_hbm, agg_hbm,
                  *rest):
        if with_counts:
            (cnt_hbm, sidx, didx, av, bv, cv8, eov, eov8, zv, agg_sp,
             sem_a, sem_b, onesv, cnt_sp) = rest
        else:
            (sidx, didx, av, bv, cv8, eov, eov8, zv, agg_sp,
             sem_a, sem_b) = rest
        cid = lax.axis_index("c")
        sid = lax.axis_index("s")
        wid = sid * _NC + cid

        @pl.loop(0, zr)
        def _zfill(j):
            zv[j] = jnp.zeros((_L,), jnp.float32)

        zoff = pl.multiple_of(sid * zr, zr)
        pltpu.sync_copy(zv, agg_sp.at[pl.ds(zoff, zr)])
        if with_counts:
            @pl.loop(0, _CH)
            def _ofill(j):
                onesv[j] = jnp.ones((_L,), jnp.float32)
            pltpu.sync_copy(zv, cnt_sp.at[pl.ds(zoff, zr)])
        plsc.subcore_barrier()

        @pl.loop(0, per_w)
        def _chunk(k):
            c = wid + k * nw

            @pl.when(c < nchunk)
            def _():
                base = pl.multiple_of(c * _CH, _CH)
                base8 = pl.multiple_of(c * chp, chp)
                pltpu.sync_copy(src_hbm.at[pl.ds(base, _CH)], sidx)
                pltpu.sync_copy(dst_hbm.at[pl.ds(base, _CH)], didx)
                ga = pltpu.async_copy(a_hbm.at[sidx], av, sem_a)
                gb = pltpu.async_copy(b_hbm.at[didx], bv, sem_b)
                pltpu.sync_copy(c_hbm.at[pl.ds(base8, chp)], cv8)
                ga.wait()
                gb.wait()

                @pl.loop(0, chp)
                def _rows(i):
                    e0 = i * 8
                    for j in range(8):
                        v = (av[e0 + j] + bv[e0 + j]
                             + cv8[i, pl.ds(j * _L, _L)])
                        eov[e0 + j] = v
                        eov8[i, pl.ds(j * _L, _L)] = v

                pltpu.sync_copy(eov8, eo_hbm.at[pl.ds(base8, chp)])
                pltpu.sync_copy(eov, agg_sp.at[didx], add=True)
                if with_counts:
                    pltpu.sync_copy(onesv, cnt_sp.at[didx], add=True)

        plsc.subcore_barrier()
        osl = pl.multiple_of(sid * zr, zr)
        ohb = pl.multiple_of(cid * npad + sid * zr, zr)
        pltpu.sync_copy(agg_sp.at[pl.ds(osl, zr)], agg_hbm.at[pl.ds(ohb, zr)])
        if with_counts:
            pltpu.sync_copy(cnt_sp.at[pl.ds(osl, zr)],
                            cnt_hbm.at[pl.ds(ohb, zr)])

    outs = sc_kernel(a_tab, b_tab, c8, src, dst)
    if with_counts:
        eo8, agg, cnt = outs
        return eo8, (agg[:n], agg[npad:npad + n]), (cnt[:n], cnt[npad:npad + n])
    eo8, agg = outs
    return eo8, (agg[:n], agg[npad:npad + n]), None


def _full(shape):
    return pl.BlockSpec(shape, lambda i: (0, 0))


def _pre_node(h_v, ws, wd, bn=1000):
    n, dv = h_v.shape
    de = ws.shape[1]

    def body(hv_ref, ws_ref, wd_ref, a_ref, b_ref):
        hv = hv_ref[...]
        a_ref[...] = jnp.dot(hv, ws_ref[...], preferred_element_type=jnp.float32)
        b_ref[...] = jnp.dot(hv, wd_ref[...], preferred_element_type=jnp.float32)

    return pl.pallas_call(
        body,
        grid=(n // bn,),
        in_specs=[pl.BlockSpec((bn, dv), lambda i: (i, 0)),
                  _full((dv, de)), _full((dv, de))],
        out_specs=[pl.BlockSpec((bn, de), lambda i: (i, 0))] * 2,
        out_shape=[jax.ShapeDtypeStruct((n, de), jnp.float32)] * 2,
    )(h_v, ws, wd)


def _edge_in(he8, k0, be_t, be_rows=2000):
    e8 = he8.shape[0]

    def body(he_ref, k_ref, be_ref, c_ref):
        c_ref[...] = (jnp.dot(he_ref[...], k_ref[...],
                              preferred_element_type=jnp.float32) + be_ref[...])

    return pl.pallas_call(
        body,
        grid=(e8 // be_rows,),
        in_specs=[pl.BlockSpec((be_rows, 128), lambda i: (i, 0)),
                  _full((128, 128)), _full((1, 128))],
        out_specs=pl.BlockSpec((be_rows, 128), lambda i: (i, 0)),
        out_shape=jax.ShapeDtypeStruct((e8, 128), jnp.float32),
    )(he8, k0, be_t)


def _edge_post(eo8, he8, m_avg, g_t, b_t, k_next=None, be_next=None,
               be_rows=2000):
    e8 = eo8.shape[0]
    has_next = k_next is not None

    def body(eo_ref, he_ref, m_ref, g_ref, b_ref, *rest):
        if has_next:
            kn_ref, ben_ref, hn_ref, cn_ref = rest
        else:
            (hn_ref,) = rest
        r = jnp.maximum(eo_ref[...], 0.0)
        mavg = m_ref[...]
        mu = jnp.dot(r, mavg, preferred_element_type=jnp.float32)
        q = r - mu
        var = jnp.dot(q * q, mavg, preferred_element_type=jnp.float32)
        ln = q * lax.rsqrt(var + 1e-5) * g_ref[...] + b_ref[...]
        hn = he_ref[...] + ln
        hn_ref[...] = hn
        if has_next:
            cn_ref[...] = (jnp.dot(hn, kn_ref[...],
                                   preferred_element_type=jnp.float32)
                           + ben_ref[...])

    in_specs = [pl.BlockSpec((be_rows, 128), lambda i: (i, 0)),
                pl.BlockSpec((be_rows, 128), lambda i: (i, 0)),
                _full((128, 128)), _full((1, 128)), _full((1, 128))]
    out_specs = [pl.BlockSpec((be_rows, 128), lambda i: (i, 0))]
    out_shape = [jax.ShapeDtypeStruct((e8, 128), jnp.float32)]
    args = [eo8, he8, m_avg, g_t, b_t]
    if has_next:
        in_specs += [_full((128, 128)), _full((1, 128))]
        out_specs = out_specs * 2
        out_shape = out_shape * 2
        args += [k_next, be_next]
    res = pl.pallas_call(
        body, grid=(e8 // be_rows,), in_specs=in_specs,
        out_specs=out_specs, out_shape=out_shape,
    )(*args)
    return res if has_next else (res[0], None)


def _node_post(h_v, agg0, agg1, cnt0, cnt1, wvh, wvm, bv, g, b,
               ws_next=None, wd_next=None, bn=1000):
    n, dv = h_v.shape
    de = agg0.shape[1]
    has_next = ws_next is not None

    def body(hv_ref, a0_ref, a1_ref, c0_ref, c1_ref, wvh_ref, wvm_ref,
             bv_ref, g_ref, b_ref, *rest):
        if has_next:
            wsn_ref, wdn_ref, hn_ref, an_ref, bn_ref = rest
        else:
            (hn_ref,) = rest
        aggt = a0_ref[...] + a1_ref[...]
        cntt = c0_ref[...] + c1_ref[...]
        mean = aggt / jnp.maximum(cntt, 1.0)
        hv = hv_ref[...]
        v = (jnp.dot(hv, wvh_ref[...], preferred_element_type=jnp.float32)
             + jnp.dot(mean, wvm_ref[...], preferred_element_type=jnp.float32)
             + bv_ref[...])
        v = jnp.maximum(v, 0.0)
        mu = jnp.mean(v, axis=-1, keepdims=True)
        q = v - mu
        var = jnp.mean(q * q, axis=-1, keepdims=True)
        ln = q * lax.rsqrt(var + 1e-5) * g_ref[...] + b_ref[...]
        hn = hv + ln
        hn_ref[...] = hn
        if has_next:
            an_ref[...] = jnp.dot(hn, wsn_ref[...],
                                  preferred_element_type=jnp.float32)
            bn_ref[...] = jnp.dot(hn, wdn_ref[...],
                                  preferred_element_type=jnp.float32)

    in_specs = [pl.BlockSpec((bn, dv), lambda i: (i, 0))] + \
               [pl.BlockSpec((bn, de), lambda i: (i, 0))] * 4 + \
               [_full((dv, dv)), _full((de, dv)), _full((1, dv)),
                _full((1, dv)), _full((1, dv))]
    out_specs = [pl.BlockSpec((bn, dv), lambda i: (i, 0))]
    out_shape = [jax.ShapeDtypeStruct((n, dv), jnp.float32)]
    args = [h_v, agg0, agg1, cnt0, cnt1, wvh, wvm, bv, g, b]
    if has_next:
        in_specs += [_full((dv, de)), _full((dv, de))]
        out_specs += [pl.BlockSpec((bn, de), lambda i: (i, 0))] * 2
        out_shape += [jax.ShapeDtypeStruct((n, de), jnp.float32)] * 2
        args += [ws_next, wd_next]
    res = pl.pallas_call(
        body, grid=(n // bn,), in_specs=in_specs,
        out_specs=out_specs, out_shape=out_shape,
    )(*args)
    return res if has_next else (res[0], None, None)


def _tile8(v):
    return jnp.tile(v, 8)[None, :]


def kernel(h_v, edge_index, h_e, params):
    n, dv = h_v.shape
    e, de = h_e.shape
    src = edge_index[0]
    dst = edge_index[1]
    he8 = h_e.reshape(e // 8, 8 * de)

    eye8 = jnp.eye(8, dtype=jnp.float32)
    m_avg = jnp.asarray(np.kron(np.eye(8, dtype=np.float32),
                                np.full((16, 16), 1.0 / 16, np.float32)))

    prep = []
    for p in params:
        prep.append({
            'ws': p['We'][:dv],
            'wd': p['We'][dv:2 * dv],
            'ke': jnp.kron(eye8, p['We'][2 * dv:]),
            'be_t': _tile8(p['be']),
            'wvh': p['Wv'][:dv],
            'wvm': p['Wv'][dv:],
            'bv': p['bv'][None, :],
            'g_v': p['g_v'][None, :],
            'b_v': p['b_v'][None, :],
            'g_e_t': _tile8(p['g_e']),
            'b_e_t': _tile8(p['b_e']),
        })

    nb = len(prep)
    a_tab, b_tab = _pre_node(h_v, prep[0]['ws'], prep[0]['wd'])
    c8 = _edge_in(he8, prep[0]['ke'], prep[0]['be_t'])
    cnt0 = cnt1 = None
    for blk in range(nb):
        p = prep[blk]
        last = blk == nb - 1
        pn = None if last else prep[blk + 1]
        eo8, (agg0, agg1), cnts = _sc_edge_pass(
            a_tab, b_tab, c8, src, dst, with_counts=(blk == 0))
        if cnts is not None:
            cnt0, cnt1 = cnts
        he8, c8 = _edge_post(
            eo8, he8, m_avg, p['g_e_t'], p['b_e_t'],
            None if last else pn['ke'], None if last else pn['be_t'])
        h_v, a_tab, b_tab = _node_post(
            h_v, agg0, agg1, cnt0, cnt1, p['wvh'], p['wvm'], p['bv'],
            p['g_v'], p['b_v'],
            None if last else pn['ws'], None if last else pn['wd'])

    return h_v, he8.reshape(e, de)


# speedup vs baseline: 9.9901x; 1.4662x over previous
import functools

import numpy as np
import jax
import jax.numpy as jnp
from jax import lax
from jax.experimental import pallas as pl
from jax.experimental.pallas import tpu as pltpu
from jax.experimental.pallas import tpu_sc as plsc

_NC = 2
_NS = 16
_L = 16
_CH = 128


_MC = 4
_ME = _MC * _CH


def _sc_edge_pass(a_tab, b_tab, c8, ei3, with_counts):
    n = a_tab.shape[0]
    e8 = c8.shape[0]
    e = e8 * 8
    nw = _NC * _NS
    nmac = e // _ME
    mep = _ME // 8
    zr = 640
    npad = zr * _NS
    per_w = (nmac + nw - 1) // nw
    per_w += per_w % 2

    out_type = [
        jax.ShapeDtypeStruct((e8, 8 * _L), jnp.float32),
        jax.ShapeDtypeStruct((_NC * npad, _L), jnp.float32),
    ]

    def bufset():
        return [
            pltpu.VMEM((2, _MC, _CH), jnp.int32),
            pltpu.VMEM((_ME, _L), jnp.float32),
            pltpu.VMEM((_ME, _L), jnp.float32),
            pltpu.VMEM((mep, 8 * _L), jnp.float32),
            pltpu.VMEM((_ME, _L), jnp.float32),
            pltpu.VMEM((mep, 8 * _L), jnp.float32),
            pltpu.SemaphoreType.DMA,
            pltpu.SemaphoreType.DMA,
            pltpu.SemaphoreType.DMA,
        ]

    scratch = bufset() + bufset() + [
        pltpu.VMEM((zr, _L), jnp.float32),
        pltpu.VMEM_SHARED((npad, _L), jnp.float32),
    ]
    if with_counts:
        out_type.append(jax.ShapeDtypeStruct((_NC * npad, _L), jnp.float32))
        scratch.append(pltpu.VMEM((_CH, _L), jnp.float32))
        scratch.append(pltpu.VMEM_SHARED((npad, _L), jnp.float32))

    mesh = plsc.VectorSubcoreMesh(core_axis_name="c", subcore_axis_name="s")

    @functools.partial(
        pl.kernel, out_type=tuple(out_type), mesh=mesh,
        scratch_types=scratch,
        compiler_params=pltpu.CompilerParams(use_tc_tiling_on_sc=False))
    def sc_kernel(a_hbm, b_hbm, c_hbm, ei_hbm, eo_hbm, agg_hbm, *rest):
        if with_counts:
            cnt_hbm = rest[0]
            rest = rest[1:]
        s0 = rest[0:9]
        s1 = rest[9:18]
        zv, agg_sp = rest[18], rest[19]
        if with_counts:
            onesv, cnt_sp = rest[20], rest[21]
        cid = lax.axis_index("c")
        sid = lax.axis_index("s")
        wid = sid * _NC + cid

        def issue_pre(m, S):
            idxb, _, _, cv8, _, _, sem_pre, _, _ = S
            mb = pl.multiple_of(m * _MC, _MC)
            pltpu.async_copy(ei_hbm.at[:, pl.ds(mb, _MC)], idxb, sem_pre)
            mb8 = pl.multiple_of(m * mep, mep)
            pltpu.async_copy(c_hbm.at[pl.ds(mb8, mep)], cv8, sem_pre)

        def wait_pre(S):
            idxb, _, _, cv8, _, _, sem_pre, _, _ = S
            pltpu.make_async_copy(ei_hbm.at[:, pl.ds(0, _MC)], idxb,
                                  sem_pre).wait()
            pltpu.make_async_copy(c_hbm.at[pl.ds(0, mep)], cv8,
                                  sem_pre).wait()

        def issue_gath(S):
            idxb, av, bv, _, _, _, _, sem_g, _ = S
            for j in range(_MC):
                pltpu.async_copy(a_hbm.at[idxb.at[0, j]],
                                 av.at[pl.ds(j * _CH, _CH)], sem_g)
                pltpu.async_copy(b_hbm.at[idxb.at[1, j]],
                                 bv.at[pl.ds(j * _CH, _CH)], sem_g)

        def wait_gath(S):
            idxb, av, bv, _, _, _, _, sem_g, _ = S
            for j in range(_MC):
                pltpu.make_async_copy(a_hbm.at[idxb.at[0, j]],
                                      av.at[pl.ds(j * _CH, _CH)],
                                      sem_g).wait()
                pltpu.make_async_copy(b_hbm.at[idxb.at[1, j]],
                                      bv.at[pl.ds(j * _CH, _CH)],
                                      sem_g).wait()

        def drain_out(S):
            _, _, _, _, _, eov8, _, _, sem_out = S
            pltpu.make_async_copy(eov8, eo_hbm.at[pl.ds(0, mep)],
                                  sem_out).wait()

        def run_macro(m, S):
            idxb, av, bv, cv8, eov, eov8, _, _, sem_out = S

            @pl.loop(0, mep)
            def _rows(i):
                e0 = i * 8
                for j in range(8):
                    v = (av[e0 + j] + bv[e0 + j]
                         + cv8[i, pl.ds(j * _L, _L)])
                    eov[e0 + j] = v
                    eov8[i, pl.ds(j * _L, _L)] = v

            mb8 = pl.multiple_of(m * mep, mep)
            pltpu.async_copy(eov8, eo_hbm.at[pl.ds(mb8, mep)], sem_out)
            for j in range(_MC):
                pltpu.sync_copy(eov.at[pl.ds(j * _CH, _CH)],
                                agg_sp.at[idxb.at[1, j]], add=True)
                if with_counts:
                    pltpu.sync_copy(onesv, cnt_sp.at[idxb.at[1, j]],
                                    add=True)

        @pl.loop(0, zr)
        def _zfill(j):
            zv[j] = jnp.zeros((_L,), jnp.float32)

        zoff = pl.multiple_of(sid * zr, zr)
        pltpu.sync_copy(zv, agg_sp.at[pl.ds(zoff, zr)])
        if with_counts:
            @pl.loop(0, _CH)
            def _ofill(j):
                onesv[j] = jnp.ones((_L,), jnp.float32)
            pltpu.sync_copy(zv, cnt_sp.at[pl.ds(zoff, zr)])
        plsc.subcore_barrier()

        issue_pre(wid, s0)
        issue_pre(wid + nw, s1)
        wait_pre(s0)
        issue_gath(s0)

        @pl.loop(0, per_w, step=2)
        def _pipe(k):
            for off, cur, nxt in ((0, s0, s1), (1, s1, s0)):
                kk = k + off
                m_cur = wid + kk * nw
                m_nxt = wid + (kk + 1) * nw
                m_pre = wid + (kk + 2) * nw

                @pl.when(m_cur < nmac)
                def _():
                    wait_gath(cur)

                    @pl.when(m_nxt < nmac)
                    def _():
                        wait_pre(nxt)
                        issue_gath(nxt)

                    @pl.when(kk >= 2)
                    def _():
                        drain_out(cur)

                    run_macro(m_cur, cur)

                    @pl.when(m_pre < nmac)
                    def _():
                        issue_pre(m_pre, cur)

        drain_out(s0)
        drain_out(s1)

        plsc.subcore_barrier()
        osl = pl.multiple_of(sid * zr, zr)
        ohb = pl.multiple_of(cid * npad + sid * zr, zr)
        pltpu.sync_copy(agg_sp.at[pl.ds(osl, zr)], agg_hbm.at[pl.ds(ohb, zr)])
        if with_counts:
            pltpu.sync_copy(cnt_sp.at[pl.ds(osl, zr)],
                            cnt_hbm.at[pl.ds(ohb, zr)])

    outs = sc_kernel(a_tab, b_tab, c8, ei3)
    if with_counts:
        eo8, agg, cnt = outs
        return eo8, (agg[:n], agg[npad:npad + n]), (cnt[:n], cnt[npad:npad + n])
    eo8, agg = outs
    return eo8, (agg[:n], agg[npad:npad + n]), None


def _full(shape):
    return pl.BlockSpec(shape, lambda i: (0, 0))


def _pre_node(h_v, ws, wd, bn=1000):
    n, dv = h_v.shape
    de = ws.shape[1]

    def body(hv_ref, ws_ref, wd_ref, a_ref, b_ref):
        hv = hv_ref[...]
        a_ref[...] = jnp.dot(hv, ws_ref[...], preferred_element_type=jnp.float32)
        b_ref[...] = jnp.dot(hv, wd_ref[...], preferred_element_type=jnp.float32)

    return pl.pallas_call(
        body,
        grid=(n // bn,),
        in_specs=[pl.BlockSpec((bn, dv), lambda i: (i, 0)),
                  _full((dv, de)), _full((dv, de))],
        out_specs=[pl.BlockSpec((bn, de), lambda i: (i, 0))] * 2,
        out_shape=[jax.ShapeDtypeStruct((n, de), jnp.float32)] * 2,
    )(h_v, ws, wd)


def _edge_in(he8, k0, be_t, be_rows=2000):
    e8 = he8.shape[0]

    def body(he_ref, k_ref, be_ref, c_ref):
        c_ref[...] = (jnp.dot(he_ref[...], k_ref[...],
                              preferred_element_type=jnp.float32) + be_ref[...])

    return pl.pallas_call(
        body,
        grid=(e8 // be_rows,),
        in_specs=[pl.BlockSpec((be_rows, 128), lambda i: (i, 0)),
                  _full((128, 128)), _full((1, 128))],
        out_specs=pl.BlockSpec((be_rows, 128), lambda i: (i, 0)),
        out_shape=jax.ShapeDtypeStruct((e8, 128), jnp.float32),
    )(he8, k0, be_t)


def _edge_post(eo8, he8, m_avg, g_t, b_t, k_next=None, be_next=None,
               be_rows=2000):
    e8 = eo8.shape[0]
    has_next = k_next is not None

    def body(eo_ref, he_ref, m_ref, g_ref, b_ref, *rest):
        if has_next:
            kn_ref, ben_ref, hn_ref, cn_ref = rest
        else:
            (hn_ref,) = rest
        r = jnp.maximum(eo_ref[...], 0.0)
        mavg = m_ref[...]
        mu = jnp.dot(r, mavg, preferred_element_type=jnp.float32)
        q = r - mu
        var = jnp.dot(q * q, mavg, preferred_element_type=jnp.float32)
        ln = q * lax.rsqrt(var + 1e-5) * g_ref[...] + b_ref[...]
        hn = he_ref[...] + ln
        hn_ref[...] = hn
        if has_next:
            cn_ref[...] = (jnp.dot(hn, kn_ref[...],
                                   preferred_element_type=jnp.float32)
                           + ben_ref[...])

    in_specs = [pl.BlockSpec((be_rows, 128), lambda i: (i, 0)),
                pl.BlockSpec((be_rows, 128), lambda i: (i, 0)),
                _full((128, 128)), _full((1, 128)), _full((1, 128))]
    out_specs = [pl.BlockSpec((be_rows, 128), lambda i: (i, 0))]
    out_shape = [jax.ShapeDtypeStruct((e8, 128), jnp.float32)]
    args = [eo8, he8, m_avg, g_t, b_t]
    if has_next:
        in_specs += [_full((128, 128)), _full((1, 128))]
        out_specs = out_specs * 2
        out_shape = out_shape * 2
        args += [k_next, be_next]
    res = pl.pallas_call(
        body, grid=(e8 // be_rows,), in_specs=in_specs,
        out_specs=out_specs, out_shape=out_shape,
    )(*args)
    return res if has_next else (res[0], None)


def _node_post(h_v, agg0, agg1, cnt0, cnt1, wvh, wvm, bv, g, b,
               ws_next=None, wd_next=None, bn=1000):
    n, dv = h_v.shape
    de = agg0.shape[1]
    has_next = ws_next is not None

    def body(hv_ref, a0_ref, a1_ref, c0_ref, c1_ref, wvh_ref, wvm_ref,
             bv_ref, g_ref, b_ref, *rest):
        if has_next:
            wsn_ref, wdn_ref, hn_ref, an_ref, bn_ref = rest
        else:
            (hn_ref,) = rest
        aggt = a0_ref[...] + a1_ref[...]
        cntt = c0_ref[...] + c1_ref[...]
        mean = aggt / jnp.maximum(cntt, 1.0)
        hv = hv_ref[...]
        v = (jnp.dot(hv, wvh_ref[...], preferred_element_type=jnp.float32)
             + jnp.dot(mean, wvm_ref[...], preferred_element_type=jnp.float32)
             + bv_ref[...])
        v = jnp.maximum(v, 0.0)
        mu = jnp.mean(v, axis=-1, keepdims=True)
        q = v - mu
        var = jnp.mean(q * q, axis=-1, keepdims=True)
        ln = q * lax.rsqrt(var + 1e-5) * g_ref[...] + b_ref[...]
        hn = hv + ln
        hn_ref[...] = hn
        if has_next:
            an_ref[...] = jnp.dot(hn, wsn_ref[...],
                                  preferred_element_type=jnp.float32)
            bn_ref[...] = jnp.dot(hn, wdn_ref[...],
                                  preferred_element_type=jnp.float32)

    in_specs = [pl.BlockSpec((bn, dv), lambda i: (i, 0))] + \
               [pl.BlockSpec((bn, de), lambda i: (i, 0))] * 4 + \
               [_full((dv, dv)), _full((de, dv)), _full((1, dv)),
                _full((1, dv)), _full((1, dv))]
    out_specs = [pl.BlockSpec((bn, dv), lambda i: (i, 0))]
    out_shape = [jax.ShapeDtypeStruct((n, dv), jnp.float32)]
    args = [h_v, agg0, agg1, cnt0, cnt1, wvh, wvm, bv, g, b]
    if has_next:
        in_specs += [_full((dv, de)), _full((dv, de))]
        out_specs += [pl.BlockSpec((bn, de), lambda i: (i, 0))] * 2
        out_shape += [jax.ShapeDtypeStruct((n, de), jnp.float32)] * 2
        args += [ws_next, wd_next]
    res = pl.pallas_call(
        body, grid=(n // bn,), in_specs=in_specs,
        out_specs=out_specs, out_shape=out_shape,
    )(*args)
    return res if has_next else (res[0], None, None)


def _tile8(v):
    return jnp.tile(v, 8)[None, :]


def kernel(h_v, edge_index, h_e, params):
    n, dv = h_v.shape
    e, de = h_e.shape
    ei3 = edge_index.reshape(2, e // _CH, _CH)
    he8 = h_e.reshape(e // 8, 8 * de)

    eye8 = jnp.eye(8, dtype=jnp.float32)
    m_avg = jnp.asarray(np.kron(np.eye(8, dtype=np.float32),
                                np.full((16, 16), 1.0 / 16, np.float32)))

    prep = []
    for p in params:
        prep.append({
            'ws': p['We'][:dv],
            'wd': p['We'][dv:2 * dv],
            'ke': jnp.kron(eye8, p['We'][2 * dv:]),
            'be_t': _tile8(p['be']),
            'wvh': p['Wv'][:dv],
            'wvm': p['Wv'][dv:],
            'bv': p['bv'][None, :],
            'g_v': p['g_v'][None, :],
            'b_v': p['b_v'][None, :],
            'g_e_t': _tile8(p['g_e']),
            'b_e_t': _tile8(p['b_e']),
        })

    nb = len(prep)
    a_tab, b_tab = _pre_node(h_v, prep[0]['ws'], prep[0]['wd'])
    c8 = _edge_in(he8, prep[0]['ke'], prep[0]['be_t'])
    cnt0 = cnt1 = None
    for blk in range(nb):
        p = prep[blk]
        last = blk == nb - 1
        pn = None if last else prep[blk + 1]
        eo8, (agg0, agg1), cnts = _sc_edge_pass(
            a_tab, b_tab, c8, ei3, with_counts=(blk == 0))
        if cnts is not None:
            cnt0, cnt1 = cnts
        he8, c8 = _edge_post(
            eo8, he8, m_avg, p['g_e_t'], p['b_e_t'],
            None if last else pn['ke'], None if last else pn['be_t'])
        h_v, a_tab, b_tab = _node_post(
            h_v, agg0, agg1, cnt0, cnt1, p['wvh'], p['wvm'], p['bv'],
            p['g_v'], p['b_v'],
            None if last else pn['ws'], None if last else pn['wd'])

    return h_v, he8.reshape(e, de)
